# baseline (speedup 1.0000x reference)
import functools

import jax
import jax.numpy as jnp
from jax.experimental import pallas as pl
from jax.experimental.pallas import tpu as pltpu

N, C, H, W = 16, 384, 32, 32
HW = H * W
CH = 192
K = HW // 4


def _score_body(x_ref, w1_ref, b1_ref, w2_ref, b2_ref, s_ref):
    X = x_ref[0]
    h = jnp.dot(w1_ref[...], X, preferred_element_type=jnp.float32)
    h = jnp.maximum(h + b1_ref[...], 0.0)
    s = jnp.dot(w2_ref[...], h, preferred_element_type=jnp.float32)
    s_ref[0] = s + b2_ref[...]


def _scores(fm3, W1, b1, W2, b2):
    return pl.pallas_call(
        _score_body,
        grid=(N,),
        in_specs=[
            pl.BlockSpec((1, C, HW), lambda n: (n, 0, 0)),
            pl.BlockSpec((CH, C), lambda n: (0, 0)),
            pl.BlockSpec((CH, 1), lambda n: (0, 0)),
            pl.BlockSpec((1, CH), lambda n: (0, 0)),
            pl.BlockSpec((1, 1), lambda n: (0, 0)),
        ],
        out_specs=pl.BlockSpec((1, 1, HW), lambda n: (n, 0, 0)),
        out_shape=jax.ShapeDtypeStruct((N, 1, HW), jnp.float32),
    )(fm3, W1, b1.reshape(CH, 1), W2, b2.reshape(1, 1))


def _icumsum(x):
    sh = 1
    while sh < HW:
        x = x + jnp.concatenate(
            [jnp.zeros((x.shape[0], sh), x.dtype), x[:, :-sh]], axis=1)
        sh *= 2
    return x


_HI = jax.lax.Precision.HIGHEST


def _topk_body(p_ref, idx_ref):
    p = p_ref[...]
    b = jax.lax.bitcast_convert_type(p, jnp.int32)
    key = jnp.where(b >= 0, b, jnp.int32(-2147483648) - b)

    lo = jnp.min(key, axis=1, keepdims=True)
    hi = jnp.max(key, axis=1, keepdims=True)

    def bis(_, lh):
        lo, hi = lh
        mid = lo + ((hi - lo + 1) >> 1)
        cnt = jnp.sum((key >= mid).astype(jnp.int32), axis=1, keepdims=True)
        ok = cnt >= K
        return jnp.where(ok, mid, lo), jnp.where(ok, hi, mid - 1)

    lo, hi = jax.lax.fori_loop(0, 31, bis, (lo, hi))
    v = lo
    gt = key > v
    eq = key == v
    ngt = jnp.sum(gt.astype(jnp.int32), axis=1, keepdims=True)
    eqc = _icumsum(eq.astype(jnp.int32))
    sel = gt | (eq & (eqc <= (K - ngt)))
    pos = _icumsum(sel.astype(jnp.int32)) - 1

    riota = jax.lax.broadcasted_iota(jnp.int32, (K, HW), 0)
    iiota = jax.lax.broadcasted_iota(jnp.int32, (1, HW), 1).astype(jnp.float32)
    eyeK = (jax.lax.broadcasted_iota(jnp.int32, (K, K), 0) ==
            jax.lax.broadcasted_iota(jnp.int32, (K, K), 1)).astype(jnp.float32)
    piota = jax.lax.broadcasted_iota(jnp.int32, (1, K), 1)
    dn_t = (((0,), (0,)), ((), ()))

    for bi in range(N):
        pos_b = pos[bi:bi + 1]
        sel_b = sel[bi:bi + 1]
        p_b = p[bi:bi + 1]
        M = ((jnp.broadcast_to(pos_b, (K, HW)) == riota)
             & jnp.broadcast_to(sel_b, (K, HW)))
        prob_c = jnp.sum(jnp.where(M, jnp.broadcast_to(p_b, (K, HW)), 0.0),
                         axis=1, keepdims=True)
        idx_c = jnp.sum(jnp.where(M, jnp.broadcast_to(iiota, (K, HW)), 0.0),
                        axis=1, keepdims=True)
        prob_r = jax.lax.dot_general(prob_c, eyeK, dn_t, precision=_HI)
        idx_r = jax.lax.dot_general(idx_c, eyeK, dn_t, precision=_HI)
        Cm = ((prob_r > prob_c)
              | ((prob_r == prob_c) & (idx_r < idx_c)))
        rank = jnp.sum(Cm.astype(jnp.int32), axis=1, keepdims=True)
        E = (jnp.broadcast_to(rank, (K, K)) == piota).astype(jnp.float32)
        out_r = jax.lax.dot_general(idx_c, E, dn_t, precision=_HI)
        idx_ref[bi:bi + 1, :] = out_r.astype(jnp.int32)


def _topk(probs):
    return pl.pallas_call(
        _topk_body,
        out_shape=jax.ShapeDtypeStruct((N, K), jnp.int32),
    )(probs)


def kernel(feature_map, W1, b1, W2, b2):
    fm3 = feature_map.reshape(N, C, HW)
    scores = _scores(fm3, W1, b1, W2, b2)
    probs = jax.nn.softplus(scores)
    idx = _topk(probs.reshape(N, HW))
    out = jnp.take_along_axis(fm3, idx[:, None, :], axis=2)
    return out[..., None]
